# baseline (speedup 1.0000x reference)
import functools

import jax
import jax.numpy as jnp
from jax.experimental import pallas as pl
from jax.experimental.pallas import tpu as pltpu

PACK = 2


def _silu(v):
    t = jnp.asarray(0.5, v.dtype) * v
    return t + t * jnp.tanh(t)


def _mbconv_kernel(x_ref, wbd_ref, b1_ref, wdd_ref, b2_ref,
                   wse1_ref, bse1_ref, wse2_ref, bse2_ref, wpbd_ref,
                   b3_ref, o_ref, halo_ref, *, K, H, W, LEFT):
    pad = (K - 1) // 2
    C2 = wbd_ref.shape[1]
    HW = H * W

    @pl.when(pl.program_id(0) == 0)
    def _zero_halo():
        halo_ref[...] = jnp.zeros_like(halo_ref)

    x = x_ref[0]

    y = jax.lax.dot_general(x.astype(jnp.bfloat16), wbd_ref[...],
                            (((0,), (0,)), ((), ())),
                            preferred_element_type=jnp.float32)
    y = _silu(y.astype(jnp.bfloat16) + b1_ref[...])

    halo_ref[pad:pad + H, LEFT:LEFT + W, :] = y.reshape(H, W, C2)

    acc = None
    for kh in range(K):
        for kw in range(K):
            col = LEFT - pad + kw
            t = halo_ref[kh:kh + H, col:col + W, :] * wdd_ref[kh, kw, :]
            acc = t if acc is None else acc + t
    z = _silu(acc + b2_ref[...])

    zf = z.reshape(HW, C2)
    ones = jnp.full((1, HW), 1.0, jnp.bfloat16)
    pooled = jax.lax.dot_general(ones, zf, (((1,), (0,)), ((), ())),
                                 preferred_element_type=jnp.float32) / HW
    h = jnp.dot(pooled, wse1_ref[...],
                preferred_element_type=jnp.float32) + bse1_ref[...]
    h = _silu(h)
    g = jnp.dot(h, wse2_ref[...],
                preferred_element_type=jnp.float32) + bse2_ref[...]
    se = 0.5 + 0.5 * jnp.tanh(0.5 * g)
    zz = zf * se.astype(jnp.bfloat16)

    ot = jax.lax.dot_general(wpbd_ref[...], zz, (((0,), (1,)), ((), ())),
                             preferred_element_type=jnp.float32)
    o_ref[0] = (ot + b3_ref[...] + x).astype(o_ref.dtype)


def _block_diag(w):
    return jnp.kron(jnp.eye(PACK, dtype=w.dtype), w)


def kernel(x, w_exp, s1, b1, w_dw, s2, b2, w_se1, b_se1, w_se2, b_se2,
           w_proj, s3, b3):
    N, Cin, H, W = x.shape
    Cexp = w_exp.shape[1]
    Cout = w_proj.shape[1]
    K = w_dw.shape[0]
    HW = H * W
    pad = (K - 1) // 2
    LEFT = max(8, 8 * pl.cdiv(pad, 8))
    Hp = H + 2 * pad
    Wp = LEFT + W + pad
    NP = N // PACK
    C2, CO2 = PACK * Cexp, PACK * Cout

    x_blk = x.reshape(NP, PACK * Cin, HW)
    t2 = lambda v: jnp.tile(v, PACK).reshape(1, -1)
    bf = jnp.bfloat16
    wbd = (_block_diag(w_exp) * t2(s1)).astype(bf)
    wse1bd = _block_diag(w_se1)
    wse2bd = _block_diag(w_se2)
    wpbd = (_block_diag(w_proj) * t2(s3)).astype(bf)
    wdd = (jnp.tile(w_dw, (1, 1, PACK)) * t2(s2)).astype(bf)
    Csq2 = wse1bd.shape[1]

    out = pl.pallas_call(
        functools.partial(_mbconv_kernel, K=K, H=H, W=W, LEFT=LEFT),
        out_shape=jax.ShapeDtypeStruct((NP, PACK * Cout, HW), x.dtype),
        grid=(NP,),
        in_specs=[
            pl.BlockSpec((1, PACK * Cin, HW), lambda n: (n, 0, 0)),
            pl.BlockSpec((PACK * Cin, C2), lambda n: (0, 0)),
            pl.BlockSpec((1, C2), lambda n: (0, 0)),
            pl.BlockSpec((K, K, C2), lambda n: (0, 0, 0)),
            pl.BlockSpec((1, C2), lambda n: (0, 0)),
            pl.BlockSpec((C2, Csq2), lambda n: (0, 0)),
            pl.BlockSpec((1, Csq2), lambda n: (0, 0)),
            pl.BlockSpec((Csq2, C2), lambda n: (0, 0)),
            pl.BlockSpec((1, C2), lambda n: (0, 0)),
            pl.BlockSpec((C2, CO2), lambda n: (0, 0)),
            pl.BlockSpec((CO2, 1), lambda n: (0, 0)),
        ],
        out_specs=pl.BlockSpec((1, CO2, HW), lambda n: (n, 0, 0)),
        scratch_shapes=[pltpu.VMEM((Hp, Wp, C2), jnp.bfloat16)],
        compiler_params=pltpu.CompilerParams(
            dimension_semantics=("arbitrary",)),
    )(x_blk, wbd, t2(b1).astype(bf), wdd, t2(b2).astype(bf),
      wse1bd, t2(b_se1), wse2bd, t2(b_se2), wpbd,
      t2(b3).reshape(CO2, 1))
    return out.reshape(N, Cout, H, W)


# speedup vs baseline: 2.0455x; 2.0455x over previous
import functools

import jax
import jax.numpy as jnp
from jax.experimental import pallas as pl
from jax.experimental.pallas import tpu as pltpu

PACK = 2


def _silu(v):
    t = 0.5 * v
    return t + t * jnp.tanh(t)


def _mbconv_kernel(x_ref, wbd_ref, b1_ref, wdd_ref, b2_ref,
                   wse1_ref, bse1_ref, wse2_ref, bse2_ref, wpbd_ref,
                   b3_ref, o_ref, halo_ref, *, K, H, W, LEFT):
    pad = (K - 1) // 2
    C2 = wbd_ref.shape[1]
    HW = H * W

    @pl.when(pl.program_id(0) == 0)
    def _zero_halo():
        halo_ref[...] = jnp.zeros_like(halo_ref)

    x = x_ref[...].reshape(x_ref.shape[0] * x_ref.shape[1], HW)

    y = jax.lax.dot_general(x, wbd_ref[...], (((0,), (0,)), ((), ())),
                            preferred_element_type=jnp.float32)
    y = _silu(y + b1_ref[...])

    halo_ref[pad:pad + H, LEFT:LEFT + W, :] = y.reshape(H, W, C2)

    acc = None
    for kh in range(K):
        for kw in range(K):
            col = LEFT - pad + kw
            t = halo_ref[kh:kh + H, col:col + W, :] * wdd_ref[kh, kw, :]
            acc = t if acc is None else acc + t
    z = _silu(acc + b2_ref[...])

    pooled = jnp.mean(z.reshape(HW, C2), axis=0, keepdims=True)
    h = jnp.dot(pooled, wse1_ref[...],
                preferred_element_type=jnp.float32) + bse1_ref[...]
    h = _silu(h)
    g = jnp.dot(h, wse2_ref[...],
                preferred_element_type=jnp.float32) + bse2_ref[...]
    se = 0.5 + 0.5 * jnp.tanh(0.5 * g)
    zz = z.reshape(HW, C2) * se

    ot = jax.lax.dot_general(wpbd_ref[...], zz, (((0,), (1,)), ((), ())),
                             preferred_element_type=jnp.float32)
    res = (ot + b3_ref[...] + x).astype(o_ref.dtype)
    o_ref[...] = res.reshape(o_ref.shape)


def _block_diag(w):
    return jnp.kron(jnp.eye(PACK, dtype=w.dtype), w)


def kernel(x, w_exp, s1, b1, w_dw, s2, b2, w_se1, b_se1, w_se2, b_se2,
           w_proj, s3, b3):
    N, Cin, H, W = x.shape
    Cexp = w_exp.shape[1]
    Cout = w_proj.shape[1]
    K = w_dw.shape[0]
    HW = H * W
    pad = (K - 1) // 2
    LEFT = max(8, 8 * pl.cdiv(pad, 8))
    Hp = H + 2 * pad
    Wp = LEFT + W + pad
    NP = N // PACK
    C2, CO2 = PACK * Cexp, PACK * Cout

    x_blk = x
    t2 = lambda v: jnp.tile(v, PACK).reshape(1, -1)
    wbd = _block_diag(w_exp) * t2(s1)
    wse1bd = _block_diag(w_se1)
    wse2bd = _block_diag(w_se2)
    wpbd = _block_diag(w_proj) * t2(s3)
    wdd = jnp.tile(w_dw, (1, 1, PACK)) * t2(s2)
    Csq2 = wse1bd.shape[1]

    out = pl.pallas_call(
        functools.partial(_mbconv_kernel, K=K, H=H, W=W, LEFT=LEFT),
        out_shape=jax.ShapeDtypeStruct((N, Cout, H, W), x.dtype),
        grid=(NP,),
        in_specs=[
            pl.BlockSpec((PACK, Cin, H, W), lambda n: (n, 0, 0, 0)),
            pl.BlockSpec((PACK * Cin, C2), lambda n: (0, 0)),
            pl.BlockSpec((1, C2), lambda n: (0, 0)),
            pl.BlockSpec((K, K, C2), lambda n: (0, 0, 0)),
            pl.BlockSpec((1, C2), lambda n: (0, 0)),
            pl.BlockSpec((C2, Csq2), lambda n: (0, 0)),
            pl.BlockSpec((1, Csq2), lambda n: (0, 0)),
            pl.BlockSpec((Csq2, C2), lambda n: (0, 0)),
            pl.BlockSpec((1, C2), lambda n: (0, 0)),
            pl.BlockSpec((C2, CO2), lambda n: (0, 0)),
            pl.BlockSpec((CO2, 1), lambda n: (0, 0)),
        ],
        out_specs=pl.BlockSpec((PACK, Cout, H, W), lambda n: (n, 0, 0, 0)),
        scratch_shapes=[pltpu.VMEM((Hp, Wp, C2), jnp.float32)],
        compiler_params=pltpu.CompilerParams(
            dimension_semantics=("arbitrary",)),
    )(x_blk, wbd, t2(b1), wdd, t2(b2),
      wse1bd, t2(b_se1), wse2bd, t2(b_se2), wpbd,
      t2(b3).reshape(CO2, 1))
    return out
